# baseline (speedup 1.0000x reference)
import jax
import jax.numpy as jnp
from jax.experimental import pallas as pl
from jax.experimental.pallas import tpu as pltpu

D_MODEL = 768
D_HID = 384
N_EXP = 64
TOK_BLOCK = 4096
NSTREAM = 2
HALF = TOK_BLOCK // NSTREAM


def _router_half(x, w1, b1, w2, b2, r0, gates_ref, idx_ref, probs_ref):
    h = jnp.dot(x, w1, preferred_element_type=jnp.float32,
                precision=jax.lax.Precision.DEFAULT)
    h = jnp.maximum(h + b1, 0.0)
    logits = jnp.dot(h, w2, preferred_element_type=jnp.float32,
                     precision=jax.lax.Precision.DEFAULT)
    logits = logits + b2
    m = jnp.max(logits, axis=-1, keepdims=True)
    e = jnp.exp(logits - m)
    s = jnp.sum(e, axis=-1, keepdims=True)
    p = e / s
    probs_ref[pl.ds(r0, HALF), :] = p

    iota = jax.lax.broadcasted_iota(jnp.int32, p.shape, 1)
    v1 = jnp.max(p, axis=-1, keepdims=True)
    i1 = jnp.min(jnp.where(p >= v1, iota, N_EXP), axis=-1, keepdims=True)
    pm = jnp.where(iota == i1, -1.0, p)
    v2 = jnp.max(pm, axis=-1, keepdims=True)
    i2 = jnp.min(jnp.where(pm >= v2, iota, N_EXP), axis=-1, keepdims=True)
    denom = v1 + v2 + 1e-8
    gates_ref[pl.ds(r0, HALF), :] = jnp.concatenate(
        [v1 / denom, v2 / denom], axis=-1)
    idx_ref[pl.ds(r0, HALF), :] = jnp.concatenate([i1, i2], axis=-1)


def _router_block(*refs):
    x_refs = refs[:NSTREAM]
    w1_ref, b1_ref, w2_ref, b2_ref, gates_ref, idx_ref, probs_ref = refs[NSTREAM:]
    w1, b1, w2, b2 = w1_ref[...], b1_ref[...], w2_ref[...], b2_ref[...]
    for s, x_ref in enumerate(x_refs):
        _router_half(x_ref[...], w1, b1, w2, b2, s * HALF,
                     gates_ref, idx_ref, probs_ref)


def kernel(hidden_states, W1, b1, W2, b2):
    n_tok = hidden_states.shape[0]
    grid = (n_tok // TOK_BLOCK,)
    b1r = b1.reshape(1, D_HID)
    b2r = b2.reshape(1, N_EXP)
    out_shapes = (
        jax.ShapeDtypeStruct((n_tok, 2), jnp.float32),
        jax.ShapeDtypeStruct((n_tok, 2), jnp.int32),
        jax.ShapeDtypeStruct((n_tok, N_EXP), jnp.float32),
    )
    full = lambda shape: pl.BlockSpec(shape, lambda i: (0, 0))
    gates, idx, probs = pl.pallas_call(
        _router_block,
        grid=grid,
        in_specs=[
            pl.BlockSpec((HALF, D_MODEL),
                         (lambda s: (lambda i: (NSTREAM * i + s, 0)))(s))
            for s in range(NSTREAM)
        ] + [
            full((D_MODEL, D_HID)),
            full((1, D_HID)),
            full((D_HID, N_EXP)),
            full((1, N_EXP)),
        ],
        out_specs=(
            pl.BlockSpec((TOK_BLOCK, 2), lambda i: (i, 0)),
            pl.BlockSpec((TOK_BLOCK, 2), lambda i: (i, 0)),
            pl.BlockSpec((TOK_BLOCK, N_EXP), lambda i: (i, 0)),
        ),
        out_shape=out_shapes,
        compiler_params=pltpu.CompilerParams(
            dimension_semantics=("parallel",),
        ),
    )(*([hidden_states] * NSTREAM), W1, b1r, W2, b2r)
    return (gates, idx, probs)


# speedup vs baseline: 1.0377x; 1.0377x over previous
import jax
import jax.numpy as jnp
from jax.experimental import pallas as pl
from jax.experimental.pallas import tpu as pltpu

D_MODEL = 768
D_HID = 384
N_EXP = 64
TOK_BLOCK = 4096
NSTREAM = 4
HALF = TOK_BLOCK // NSTREAM


def _router_half(x, w1, b1, w2, b2, r0, gates_ref, idx_ref, probs_ref):
    h = jnp.dot(x, w1, preferred_element_type=jnp.float32,
                precision=jax.lax.Precision.DEFAULT)
    h = jnp.maximum(h + b1, 0.0)
    logits = jnp.dot(h, w2, preferred_element_type=jnp.float32,
                     precision=jax.lax.Precision.DEFAULT)
    logits = logits + b2
    m = jnp.max(logits, axis=-1, keepdims=True)
    e = jnp.exp(logits - m)
    s = jnp.sum(e, axis=-1, keepdims=True)
    p = e / s
    probs_ref[pl.ds(r0, HALF), :] = p

    iota = jax.lax.broadcasted_iota(jnp.int32, p.shape, 1)
    v1 = jnp.max(p, axis=-1, keepdims=True)
    i1 = jnp.min(jnp.where(p >= v1, iota, N_EXP), axis=-1, keepdims=True)
    pm = jnp.where(iota == i1, -1.0, p)
    v2 = jnp.max(pm, axis=-1, keepdims=True)
    i2 = jnp.min(jnp.where(pm >= v2, iota, N_EXP), axis=-1, keepdims=True)
    denom = v1 + v2 + 1e-8
    gates_ref[pl.ds(r0, HALF), :] = jnp.concatenate(
        [v1 / denom, v2 / denom], axis=-1)
    idx_ref[pl.ds(r0, HALF), :] = jnp.concatenate([i1, i2], axis=-1)


def _router_block(*refs):
    x_refs = refs[:NSTREAM]
    w1_ref, b1_ref, w2_ref, b2_ref, gates_ref, idx_ref, probs_ref = refs[NSTREAM:]
    w1, b1, w2, b2 = w1_ref[...], b1_ref[...], w2_ref[...], b2_ref[...]
    for s, x_ref in enumerate(x_refs):
        _router_half(x_ref[...], w1, b1, w2, b2, s * HALF,
                     gates_ref, idx_ref, probs_ref)


def kernel(hidden_states, W1, b1, W2, b2):
    n_tok = hidden_states.shape[0]
    grid = (n_tok // TOK_BLOCK,)
    b1r = b1.reshape(1, D_HID)
    b2r = b2.reshape(1, N_EXP)
    out_shapes = (
        jax.ShapeDtypeStruct((n_tok, 2), jnp.float32),
        jax.ShapeDtypeStruct((n_tok, 2), jnp.int32),
        jax.ShapeDtypeStruct((n_tok, N_EXP), jnp.float32),
    )
    full = lambda shape: pl.BlockSpec(shape, lambda i: (0, 0))
    gates, idx, probs = pl.pallas_call(
        _router_block,
        grid=grid,
        in_specs=[
            pl.BlockSpec((HALF, D_MODEL),
                         (lambda s: (lambda i: (NSTREAM * i + s, 0)))(s))
            for s in range(NSTREAM)
        ] + [
            full((D_MODEL, D_HID)),
            full((1, D_HID)),
            full((D_HID, N_EXP)),
            full((1, N_EXP)),
        ],
        out_specs=(
            pl.BlockSpec((TOK_BLOCK, 2), lambda i: (i, 0)),
            pl.BlockSpec((TOK_BLOCK, 2), lambda i: (i, 0)),
            pl.BlockSpec((TOK_BLOCK, N_EXP), lambda i: (i, 0)),
        ),
        out_shape=out_shapes,
        compiler_params=pltpu.CompilerParams(
            dimension_semantics=("parallel",),
        ),
    )(*([hidden_states] * NSTREAM), W1, b1r, W2, b2r)
    return (gates, idx, probs)


# speedup vs baseline: 1.0379x; 1.0002x over previous
import jax
import jax.numpy as jnp
from jax.experimental import pallas as pl
from jax.experimental.pallas import tpu as pltpu

D_MODEL = 768
D_HID = 384
N_EXP = 64
TOK_BLOCK = 4096
NSTREAM = 4
HALF = TOK_BLOCK // NSTREAM


def _router_half(x, w1, b1, w2, b2, r0, gates_ref, idx_ref, probs_ref):
    h = jnp.dot(x, w1, preferred_element_type=jnp.float32,
                precision=jax.lax.Precision.DEFAULT)
    h = jnp.maximum(h + b1, 0.0)
    logits = jnp.dot(h, w2, preferred_element_type=jnp.float32,
                     precision=jax.lax.Precision.DEFAULT)
    logits = logits + b2
    m = jnp.max(logits, axis=-1, keepdims=True)
    e = jnp.exp(logits - m)
    s = jnp.sum(e, axis=-1, keepdims=True)
    p = e / s
    probs_ref[pl.ds(r0, HALF), :] = p

    iota = jax.lax.broadcasted_iota(jnp.int32, p.shape, 1)
    v1 = jnp.max(p, axis=-1, keepdims=True)
    i1 = jnp.min(jnp.where(p >= v1, iota, N_EXP), axis=-1, keepdims=True)
    pm = jnp.where(iota == i1, -1.0, p)
    v2 = jnp.max(pm, axis=-1, keepdims=True)
    i2 = jnp.min(jnp.where(pm >= v2, iota, N_EXP), axis=-1, keepdims=True)
    denom = v1 + v2 + 1e-8
    gates_ref[pl.ds(r0, HALF), :] = jnp.concatenate(
        [v1 / denom, v2 / denom], axis=-1)
    idx_ref[pl.ds(r0, HALF), :] = jnp.concatenate([i1, i2], axis=-1)


def _router_block(*refs):
    x_refs = refs[:NSTREAM]
    w1_ref, b1_ref, w2_ref, b2_ref, gates_ref, idx_ref, probs_ref = refs[NSTREAM:]
    w1, b1, w2, b2 = w1_ref[...], b1_ref[...], w2_ref[...], b2_ref[...]
    for s, x_ref in enumerate(x_refs):
        _router_half(x_ref[...], w1, b1, w2, b2, s * HALF,
                     gates_ref, idx_ref, probs_ref)


def kernel(hidden_states, W1, b1, W2, b2):
    n_tok = hidden_states.shape[0]
    grid = (n_tok // TOK_BLOCK,)
    b1r = b1.reshape(1, D_HID)
    b2r = b2.reshape(1, N_EXP)
    out_shapes = (
        jax.ShapeDtypeStruct((n_tok, 2), jnp.float32),
        jax.ShapeDtypeStruct((n_tok, 2), jnp.int32),
        jax.ShapeDtypeStruct((n_tok, N_EXP), jnp.float32),
    )
    full = lambda shape: pl.BlockSpec(shape, lambda i: (0, 0))
    gates, idx, probs = pl.pallas_call(
        _router_block,
        grid=grid,
        in_specs=[
            pl.BlockSpec((HALF, D_MODEL),
                         (lambda s: (lambda i: (NSTREAM * i + s, 0)))(s))
            for s in range(NSTREAM)
        ] + [
            full((D_MODEL, D_HID)),
            full((1, D_HID)),
            full((D_HID, N_EXP)),
            full((1, N_EXP)),
        ],
        out_specs=(
            pl.BlockSpec((TOK_BLOCK, 2), lambda i: (i, 0)),
            pl.BlockSpec((TOK_BLOCK, 2), lambda i: (i, 0)),
            pl.BlockSpec((TOK_BLOCK, N_EXP), lambda i: (i, 0)),
        ),
        out_shape=out_shapes,
        compiler_params=pltpu.CompilerParams(
            dimension_semantics=("arbitrary",),
        ),
    )(*([hidden_states] * NSTREAM), W1, b1r, W2, b2r)
    return (gates, idx, probs)
